# baseline (speedup 1.0000x reference)
import jax
import jax.numpy as jnp
from jax import lax
from jax.experimental import pallas as pl
from jax.experimental.pallas import tpu as pltpu
from jax.experimental.pallas import tpu_sc as plsc

B = 16384
EMB = 32
NC = 2
NS = 16
L = 16
NW = NC * NS
BPW = B // NW
PASS = 128
NP = BPW // PASS
GP = PASS // L
ROW = 128
UPR = ROW // EMB
BIAS_PAD = 782 * ROW


def _sc_body(users_hbm, items_hbm, ut_hbm, it_hbm, bt_hbm, out_hbm,
             uidx_v, iidx_v, gu_v, gi_v, gb_v, ubuf, ibuf, bbuf, out_v, sems):
    wid = lax.axis_index("s") * NC + lax.axis_index("c")
    base = wid * BPW

    pltpu.sync_copy(users_hbm.at[pl.ds(base, BPW)], uidx_v)
    pltpu.sync_copy(items_hbm.at[pl.ds(base, BPW)], iidx_v)

    def shift_chunk(c, carry):
        sl = pl.ds(c * L, L)
        u = uidx_v[sl]
        it = iidx_v[sl]
        gu_v[sl] = lax.shift_right_logical(u, 2)
        gi_v[sl] = lax.shift_right_logical(it, 2)
        gb_v[sl] = lax.shift_right_logical(it, 7)
        return carry

    lax.fori_loop(0, BPW // L, shift_chunk, 0)

    def fire(p):
        s = p % 2
        sl = pl.ds(p * PASS, PASS)
        return [
            pltpu.async_copy(ut_hbm.at[gu_v.at[sl]], ubuf.at[s], sems.at[s]),
            pltpu.async_copy(it_hbm.at[gi_v.at[sl]], ibuf.at[s], sems.at[s]),
            pltpu.async_copy(bt_hbm.at[gb_v.at[sl]], bbuf.at[s], sems.at[s]),
        ]

    lanes = lax.iota(jnp.int32, 16)

    def compute(p):
        s = p % 2
        emb_mask = jnp.full((L,), UPR - 1, jnp.int32)
        bias_mask = jnp.full((L,), ROW - 1, jnp.int32)

        def group(g, carry):
            off = p * PASS + g * L
            uc = uidx_v[pl.ds(off, L)]
            ic = iidx_v[pl.ds(off, L)]
            rows = g * L + lanes
            ucol0 = (uc & emb_mask) * EMB
            icol0 = (ic & emb_mask) * EMB
            acc = jnp.zeros((L,), jnp.float32)
            for d in range(EMB):
                uv = plsc.load_gather(ubuf.at[s], [rows, ucol0 + d])
                iv = plsc.load_gather(ibuf.at[s], [rows, icol0 + d])
                acc = acc + uv * iv
            bv = plsc.load_gather(bbuf.at[s], [rows, ic & bias_mask])
            out_v[pl.ds(off, L)] = acc + bv
            return carry

        lax.fori_loop(0, GP, group, 0)

    pending = {}
    pending[0] = fire(0)
    pending[1] = fire(1)
    for p in range(NP):
        for cp in pending.pop(p):
            cp.wait()
        compute(p)
        if p + 2 < NP:
            pending[p + 2] = fire(p + 2)

    pltpu.sync_copy(out_v, out_hbm.at[pl.ds(base, BPW)])


def kernel(users, items, user_table, item_table, bias_table):
    n_users, emb = user_table.shape
    n_items = item_table.shape[0]
    mesh = plsc.VectorSubcoreMesh(core_axis_name="c", subcore_axis_name="s")
    f = pl.kernel(
        _sc_body,
        out_type=jax.ShapeDtypeStruct((B,), jnp.float32),
        mesh=mesh,
        compiler_params=pltpu.CompilerParams(needs_layout_passes=False),
        scratch_types=[
            pltpu.VMEM((BPW,), jnp.int32),
            pltpu.VMEM((BPW,), jnp.int32),
            pltpu.VMEM((BPW,), jnp.int32),
            pltpu.VMEM((BPW,), jnp.int32),
            pltpu.VMEM((BPW,), jnp.int32),
            pltpu.VMEM((2, PASS, ROW), jnp.float32),
            pltpu.VMEM((2, PASS, ROW), jnp.float32),
            pltpu.VMEM((2, PASS, ROW), jnp.float32),
            pltpu.VMEM((BPW,), jnp.float32),
            pltpu.SemaphoreType.DMA((2,)),
        ],
    )
    ut128 = jnp.reshape(user_table, (n_users * emb // ROW, ROW))
    it128 = jnp.reshape(item_table, (n_items * emb // ROW, ROW))
    bias_flat = jnp.reshape(bias_table, (-1,))
    bias_pad = jnp.concatenate(
        [bias_flat, jnp.zeros((BIAS_PAD - n_items,), jnp.float32)])
    bt128 = jnp.reshape(bias_pad, (BIAS_PAD // ROW, ROW))
    return f(users.astype(jnp.int32), items.astype(jnp.int32),
             ut128, it128, bt128)
